# baseline (speedup 1.0000x reference)
import functools

import jax
import jax.numpy as jnp
from jax import lax
from jax.experimental import pallas as pl
from jax.experimental.pallas import tpu as pltpu
from jax.experimental.pallas import tpu_sc as plsc

N = 10000
NNZ = 320000
D = 128
Q = 32
NQ = 4
EPS = 1e-5

NC = 2
NS = 16
R = 10240
RPT = R // NS
DUMMY = N
CH = 256
NCHUNK = 80
TOT = NS * NCHUNK * CH

_SC_PARAMS = pltpu.CompilerParams(use_tc_tiling_on_sc=False)


@functools.cache
def _mesh():
    return plsc.VectorSubcoreMesh(core_axis_name="c", subcore_axis_name="s",
                                  num_cores=NC, num_subcores=NS)


def _deg_body(n3_hbm, e3_hbm, z16_hbm, ones_hbm, binv_hbm, dinv_hbm,
              nidx_v, eidx_v, ones_v, deg_v, bd_v, deg):
    c = lax.axis_index("c")
    s = lax.axis_index("s")
    r0 = s * RPT

    pltpu.sync_copy(z16_hbm, deg.at[pl.ds(r0, RPT)])
    pltpu.sync_copy(n3_hbm.at[s], nidx_v)
    pltpu.sync_copy(e3_hbm.at[s], eidx_v)
    pltpu.sync_copy(ones_hbm, ones_v)
    plsc.subcore_barrier()

    @pl.loop(0, NCHUNK)
    def _hist_e(j):
        pltpu.sync_copy(ones_v, deg.at[eidx_v.at[j]], add=True)

    plsc.subcore_barrier()

    pltpu.sync_copy(deg.at[pl.ds(r0, RPT)], deg_v)

    @pl.loop(0, RPT)
    def _binv(i):
        cnt = deg_v[i]
        bd_v[i] = jnp.where(cnt > 0.0, 1.0 / cnt, 0.0)

    pltpu.sync_copy(bd_v, binv_hbm.at[c, pl.ds(r0, RPT)])
    pltpu.sync_copy(z16_hbm, deg.at[pl.ds(r0, RPT)])
    plsc.subcore_barrier()

    @pl.loop(0, NCHUNK)
    def _hist_n(j):
        pltpu.sync_copy(ones_v, deg.at[nidx_v.at[j]], add=True)

    plsc.subcore_barrier()

    pltpu.sync_copy(deg.at[pl.ds(r0, RPT)], deg_v)

    @pl.loop(0, RPT)
    def _dinv(i):
        cnt = deg_v[i]
        bd_v[i] = jnp.where(cnt > 0.0, 1.0 / cnt, 0.0)

    pltpu.sync_copy(bd_v, dinv_hbm.at[c, pl.ds(r0, RPT)])


def _deg(n3, e3, z16, ones):
    f = pl.kernel(
        _deg_body,
        out_type=(
            jax.ShapeDtypeStruct((NC, R, 16), jnp.float32),
            jax.ShapeDtypeStruct((NC, R, 16), jnp.float32),
        ),
        mesh=_mesh(),
        compiler_params=_SC_PARAMS,
        scratch_types=[
            pltpu.VMEM((NCHUNK, CH), jnp.int32),
            pltpu.VMEM((NCHUNK, CH), jnp.int32),
            pltpu.VMEM((CH, 16), jnp.float32),
            pltpu.VMEM((RPT, 16), jnp.float32),
            pltpu.VMEM((RPT, 16), jnp.float32),
            pltpu.VMEM_SHARED((R, 16), jnp.float32),
        ],
    )
    return f(n3, e3, z16, ones)


def _mm1_body(x_ref, w_ref, o_ref):
    o_ref[...] = lax.dot_general(
        x_ref[...], w_ref[0],
        (((1,), (1,)), ((), ())),
        preferred_element_type=jnp.float32,
    )[None]


def _mm1(xp, w1q):
    return pl.pallas_call(
        _mm1_body,
        grid=(8, NQ),
        in_specs=[
            pl.BlockSpec((R // 8, D), lambda i, j: (i, 0)),
            pl.BlockSpec((1, Q, D), lambda i, j: (j, 0, 0)),
        ],
        out_specs=pl.BlockSpec((1, R // 8, Q), lambda i, j: (j, i, 0)),
        out_shape=jax.ShapeDtypeStruct((NQ, R, Q), jnp.float32),
    )(xp, w1q)


def _hops(tbl_hbm, gidx_v, sidx_v, acc, bufs, gs, ss):
    nb = len(bufs)
    for u in range(nb):
        pltpu.async_copy(tbl_hbm.at[gidx_v.at[u]], bufs[u], gs[u])

    @pl.loop(0, NCHUNK // nb)
    def _(jq):
        base = nb * jq
        for u in range(nb):
            pltpu.make_async_copy(tbl_hbm.at[gidx_v.at[base + u]], bufs[u],
                                  gs[u]).wait()
            pltpu.async_copy(bufs[u], acc.at[sidx_v.at[base + u]], ss[u],
                             add=True)
        for u in range(nb):
            nxt = base + nb + u

            @pl.when(nxt < NCHUNK)
            def _():
                pltpu.make_async_copy(bufs[u], acc.at[sidx_v.at[base + u]],
                                      ss[u]).wait()
                pltpu.async_copy(tbl_hbm.at[gidx_v.at[nxt]], bufs[u], gs[u])

    for u in range(nb):
        pltpu.make_async_copy(bufs[u], acc.at[sidx_v.at[NCHUNK - nb + u]],
                              ss[u]).wait()


def _prop_body(h_hbm, n3_hbm, e3_hbm, z32_hbm, binv_hbm, dinv_hbm,
               out_hbm, he_hbm,
               nidx_v, eidx_v, b0, b1, rng_v, bd_v, bd2_v,
               gs0, gs1, ss0, ss1, acc):
    bufs = (b0, b1)
    gs = (gs0, gs1)
    ss = (ss0, ss1)
    c = lax.axis_index("c")
    s = lax.axis_index("s")
    r0 = s * RPT

    pltpu.sync_copy(n3_hbm.at[s], nidx_v)
    pltpu.sync_copy(e3_hbm.at[s], eidx_v)
    pltpu.sync_copy(binv_hbm.at[c, pl.ds(r0, RPT)], bd_v)
    pltpu.sync_copy(dinv_hbm.at[c, pl.ds(r0, RPT)], bd2_v)

    for p in range(2):
        fq = c * 2 + p

        pltpu.sync_copy(z32_hbm, acc.at[pl.ds(r0, RPT)])
        plsc.subcore_barrier()

        _hops(h_hbm.at[fq], nidx_v, eidx_v, acc, bufs, gs, ss)
        plsc.subcore_barrier()

        pltpu.sync_copy(acc.at[pl.ds(r0, RPT)], rng_v)

        @pl.loop(0, RPT)
        def _scale(i):
            inv = bd_v[i]
            for k in range(2):
                sl = (i, pl.ds(k * 16, 16))
                rng_v[sl] = rng_v[sl] * inv

        pltpu.sync_copy(rng_v, he_hbm.at[fq, pl.ds(r0, RPT)])
        pltpu.sync_copy(z32_hbm, acc.at[pl.ds(r0, RPT)])
        plsc.subcore_barrier()

        _hops(he_hbm.at[fq], eidx_v, nidx_v, acc, bufs, gs, ss)
        plsc.subcore_barrier()

        pltpu.sync_copy(acc.at[pl.ds(r0, RPT)], rng_v)

        @pl.loop(0, RPT)
        def _epi(i):
            dinv = bd2_v[i]
            for k in range(2):
                sl = (i, pl.ds(k * 16, 16))
                rng_v[sl] = rng_v[sl] * dinv

        pltpu.sync_copy(rng_v, out_hbm.at[fq, pl.ds(r0, RPT)])


def _prop(h, n3, e3, z32, binv, dinv):
    f = pl.kernel(
        _prop_body,
        out_type=(
            jax.ShapeDtypeStruct((NQ, R, Q), jnp.float32),
            jax.ShapeDtypeStruct((NQ, R, Q), jnp.float32),
        ),
        mesh=_mesh(),
        compiler_params=_SC_PARAMS,
        scratch_types=[
            pltpu.VMEM((NCHUNK, CH), jnp.int32),
            pltpu.VMEM((NCHUNK, CH), jnp.int32),
            pltpu.VMEM((CH, Q), jnp.float32),
            pltpu.VMEM((CH, Q), jnp.float32),
            pltpu.VMEM((RPT, Q), jnp.float32),
            pltpu.VMEM((RPT, 16), jnp.float32),
            pltpu.VMEM((RPT, 16), jnp.float32),
            pltpu.SemaphoreType.DMA,
            pltpu.SemaphoreType.DMA,
            pltpu.SemaphoreType.DMA,
            pltpu.SemaphoreType.DMA,
            pltpu.VMEM_SHARED((R, Q), jnp.float32),
        ],
    )
    return f(h, n3, e3, z32, binv, dinv)


def _bn_body(p_ref, s_ref, t_ref, o_ref):
    o_ref[...] = jnp.maximum(p_ref[...] * s_ref[...] + t_ref[...], 0.0)


def _bn(p1, s1q, t1q):
    return pl.pallas_call(
        _bn_body,
        grid=(8,),
        in_specs=[
            pl.BlockSpec((NQ, R // 8, Q), lambda i: (0, i, 0)),
            pl.BlockSpec((NQ, 1, Q), lambda i: (0, 0, 0)),
            pl.BlockSpec((NQ, 1, Q), lambda i: (0, 0, 0)),
        ],
        out_specs=pl.BlockSpec((NQ, R // 8, Q), lambda i: (0, i, 0)),
        out_shape=jax.ShapeDtypeStruct((NQ, R, Q), jnp.float32),
    )(p1, s1q, t1q)


def _mm2_body(q_ref, w2_ref, s2_ref, t2_ref, wl_ref, bl_ref, o_ref, acc_ref):
    i = pl.program_id(0)

    @pl.when(i == 0)
    def _():
        acc_ref[...] = jnp.zeros_like(acc_ref)

    o = lax.dot_general(q_ref[0], w2_ref[0], (((1,), (1,)), ((), ())),
                        preferred_element_type=jnp.float32)
    for f in range(1, NQ):
        o = o + lax.dot_general(q_ref[f], w2_ref[f], (((1,), (1,)), ((), ())),
                                preferred_element_type=jnp.float32)
    o = jnp.maximum(o * s2_ref[...] + t2_ref[...], 0.0)
    acc_ref[...] += jnp.sum(o, axis=0, keepdims=True)

    @pl.when(i == pl.num_programs(0) - 1)
    def _():
        m = acc_ref[...] * (1.0 / N)
        o_ref[...] = lax.dot_general(m, wl_ref[...], (((1,), (1,)), ((), ())),
                                     preferred_element_type=jnp.float32
                                     ) + bl_ref[...]


def _mm2(q, w2q, s2, t2, wl, bl2):
    blk = 2000
    return pl.pallas_call(
        _mm2_body,
        grid=(N // blk,),
        in_specs=[
            pl.BlockSpec((NQ, blk, Q), lambda i: (0, i, 0)),
            pl.BlockSpec((NQ, D, Q), lambda i: (0, 0, 0)),
            pl.BlockSpec((1, D), lambda i: (0, 0)),
            pl.BlockSpec((1, D), lambda i: (0, 0)),
            pl.BlockSpec((D, D), lambda i: (0, 0)),
            pl.BlockSpec((1, D), lambda i: (0, 0)),
        ],
        out_specs=pl.BlockSpec((1, D), lambda i: (0, 0)),
        out_shape=jax.ShapeDtypeStruct((1, D), jnp.float32),
        scratch_shapes=[pltpu.VMEM((1, D), jnp.float32)],
    )(q, w2q, s2, t2, wl, bl2)


def kernel(x, hyperedge_index, W1, b1, g1, be1, W2, b2, g2, be2, Wl, bl):
    f32 = jnp.float32
    xp = jnp.concatenate([x, jnp.zeros((R - N, D), f32)], axis=0)
    w1q = W1.reshape(NQ, Q, D)

    ni = hyperedge_index[0]
    ei = hyperedge_index[1]
    n3 = jnp.full((TOT,), DUMMY, jnp.int32).at[:NNZ].set(ni).reshape(
        NS, NCHUNK, CH)
    e3 = jnp.full((TOT,), DUMMY, jnp.int32).at[:NNZ].set(ei).reshape(
        NS, NCHUNK, CH)

    z32 = jnp.zeros((RPT, Q), f32)
    z16 = jnp.zeros((RPT, 16), f32)
    ones = jnp.ones((CH, 16), f32)

    c0 = 1.0 / (1.0 + EPS) ** 0.5
    s1q = (g1 * c0).reshape(NQ, 1, Q)
    t1q = ((g1 * c0) * b1 + be1).reshape(NQ, 1, Q)
    s2 = (g2 * c0).reshape(1, D)
    t2 = ((g2 * c0) * b2 + be2).reshape(1, D)
    w2q = jnp.transpose(W2.reshape(D, NQ, Q), (1, 0, 2))

    binv, dinv = _deg(n3, e3, z16, ones)
    h1 = _mm1(xp, w1q)
    p1, _ = _prop(h1, n3, e3, z32, binv, dinv)
    z1 = _bn(p1, s1q, t1q)
    q, _ = _prop(z1, n3, e3, z32, binv, dinv)
    out = _mm2(q, w2q, s2, t2, Wl, bl.reshape(1, D))
    return out[0]


# speedup vs baseline: 1.0025x; 1.0025x over previous
import functools

import jax
import jax.numpy as jnp
from jax import lax
from jax.experimental import pallas as pl
from jax.experimental.pallas import tpu as pltpu
from jax.experimental.pallas import tpu_sc as plsc

N = 10000
NNZ = 320000
D = 128
Q = 32
NQ = 4
EPS = 1e-5

NC = 2
NS = 16
R = 10240
RPT = R // NS
DUMMY = N
CH = 128
NCHUNK = 160
TOT = NS * NCHUNK * CH

_SC_PARAMS = pltpu.CompilerParams(use_tc_tiling_on_sc=False)


@functools.cache
def _mesh():
    return plsc.VectorSubcoreMesh(core_axis_name="c", subcore_axis_name="s",
                                  num_cores=NC, num_subcores=NS)


def _deg_body(n3_hbm, e3_hbm, z16_hbm, ones_hbm, binv_hbm, dinv_hbm,
              nidx_v, eidx_v, ones_v, deg_v, bd_v, deg):
    c = lax.axis_index("c")
    s = lax.axis_index("s")
    r0 = s * RPT

    pltpu.sync_copy(z16_hbm, deg.at[pl.ds(r0, RPT)])
    pltpu.sync_copy(n3_hbm.at[s], nidx_v)
    pltpu.sync_copy(e3_hbm.at[s], eidx_v)
    pltpu.sync_copy(ones_hbm, ones_v)
    plsc.subcore_barrier()

    @pl.loop(0, NCHUNK)
    def _hist_e(j):
        pltpu.sync_copy(ones_v, deg.at[eidx_v.at[j]], add=True)

    plsc.subcore_barrier()

    pltpu.sync_copy(deg.at[pl.ds(r0, RPT)], deg_v)

    @pl.loop(0, RPT)
    def _binv(i):
        cnt = deg_v[i]
        bd_v[i] = jnp.where(cnt > 0.0, 1.0 / cnt, 0.0)

    pltpu.sync_copy(bd_v, binv_hbm.at[c, pl.ds(r0, RPT)])
    pltpu.sync_copy(z16_hbm, deg.at[pl.ds(r0, RPT)])
    plsc.subcore_barrier()

    @pl.loop(0, NCHUNK)
    def _hist_n(j):
        pltpu.sync_copy(ones_v, deg.at[nidx_v.at[j]], add=True)

    plsc.subcore_barrier()

    pltpu.sync_copy(deg.at[pl.ds(r0, RPT)], deg_v)

    @pl.loop(0, RPT)
    def _dinv(i):
        cnt = deg_v[i]
        bd_v[i] = jnp.where(cnt > 0.0, 1.0 / cnt, 0.0)

    pltpu.sync_copy(bd_v, dinv_hbm.at[c, pl.ds(r0, RPT)])


def _deg(n3, e3, z16, ones):
    f = pl.kernel(
        _deg_body,
        out_type=(
            jax.ShapeDtypeStruct((NC, R, 16), jnp.float32),
            jax.ShapeDtypeStruct((NC, R, 16), jnp.float32),
        ),
        mesh=_mesh(),
        compiler_params=_SC_PARAMS,
        scratch_types=[
            pltpu.VMEM((NCHUNK, CH), jnp.int32),
            pltpu.VMEM((NCHUNK, CH), jnp.int32),
            pltpu.VMEM((CH, 16), jnp.float32),
            pltpu.VMEM((RPT, 16), jnp.float32),
            pltpu.VMEM((RPT, 16), jnp.float32),
            pltpu.VMEM_SHARED((R, 16), jnp.float32),
        ],
    )
    return f(n3, e3, z16, ones)


def _mm1_body(x_ref, w_ref, o_ref):
    o_ref[...] = lax.dot_general(
        x_ref[...], w_ref[0],
        (((1,), (1,)), ((), ())),
        preferred_element_type=jnp.float32,
    )[None]


def _mm1(xp, w1q):
    return pl.pallas_call(
        _mm1_body,
        grid=(8, NQ),
        in_specs=[
            pl.BlockSpec((R // 8, D), lambda i, j: (i, 0)),
            pl.BlockSpec((1, Q, D), lambda i, j: (j, 0, 0)),
        ],
        out_specs=pl.BlockSpec((1, R // 8, Q), lambda i, j: (j, i, 0)),
        out_shape=jax.ShapeDtypeStruct((NQ, R, Q), jnp.float32),
    )(xp, w1q)


def _hops(tbl_hbm, gidx_v, sidx_v, acc, bufs, gs, ss):
    nb = len(bufs)
    for u in range(nb):
        pltpu.async_copy(tbl_hbm.at[gidx_v.at[u]], bufs[u], gs[u])

    @pl.loop(0, NCHUNK // nb)
    def _(jq):
        base = nb * jq
        for u in range(nb):
            pltpu.make_async_copy(tbl_hbm.at[gidx_v.at[base + u]], bufs[u],
                                  gs[u]).wait()
            pltpu.async_copy(bufs[u], acc.at[sidx_v.at[base + u]], ss[u],
                             add=True)
        for u in range(nb):
            nxt = base + nb + u

            @pl.when(nxt < NCHUNK)
            def _():
                pltpu.make_async_copy(bufs[u], acc.at[sidx_v.at[base + u]],
                                      ss[u]).wait()
                pltpu.async_copy(tbl_hbm.at[gidx_v.at[nxt]], bufs[u], gs[u])

    for u in range(nb):
        pltpu.make_async_copy(bufs[u], acc.at[sidx_v.at[NCHUNK - nb + u]],
                              ss[u]).wait()


def _prop_body(h_hbm, n3_hbm, e3_hbm, z32_hbm, binv_hbm, dinv_hbm,
               out_hbm, he_hbm,
               nidx_v, eidx_v, b0, b1, b2, b3, rng_v, bd_v, bd2_v,
               gs0, gs1, gs2, gs3, ss0, ss1, ss2, ss3, acc):
    bufs = (b0, b1, b2, b3)
    gs = (gs0, gs1, gs2, gs3)
    ss = (ss0, ss1, ss2, ss3)
    c = lax.axis_index("c")
    s = lax.axis_index("s")
    r0 = s * RPT

    pltpu.sync_copy(n3_hbm.at[s], nidx_v)
    pltpu.sync_copy(e3_hbm.at[s], eidx_v)
    pltpu.sync_copy(binv_hbm.at[c, pl.ds(r0, RPT)], bd_v)
    pltpu.sync_copy(dinv_hbm.at[c, pl.ds(r0, RPT)], bd2_v)

    for p in range(2):
        fq = c * 2 + p

        pltpu.sync_copy(z32_hbm, acc.at[pl.ds(r0, RPT)])
        plsc.subcore_barrier()

        _hops(h_hbm.at[fq], nidx_v, eidx_v, acc, bufs, gs, ss)
        plsc.subcore_barrier()

        pltpu.sync_copy(acc.at[pl.ds(r0, RPT)], rng_v)

        @pl.loop(0, RPT)
        def _scale(i):
            inv = bd_v[i]
            for k in range(2):
                sl = (i, pl.ds(k * 16, 16))
                rng_v[sl] = rng_v[sl] * inv

        pltpu.sync_copy(rng_v, he_hbm.at[fq, pl.ds(r0, RPT)])
        pltpu.sync_copy(z32_hbm, acc.at[pl.ds(r0, RPT)])
        plsc.subcore_barrier()

        _hops(he_hbm.at[fq], eidx_v, nidx_v, acc, bufs, gs, ss)
        plsc.subcore_barrier()

        pltpu.sync_copy(acc.at[pl.ds(r0, RPT)], rng_v)

        @pl.loop(0, RPT)
        def _epi(i):
            dinv = bd2_v[i]
            for k in range(2):
                sl = (i, pl.ds(k * 16, 16))
                rng_v[sl] = rng_v[sl] * dinv

        pltpu.sync_copy(rng_v, out_hbm.at[fq, pl.ds(r0, RPT)])


def _prop(h, n3, e3, z32, binv, dinv):
    f = pl.kernel(
        _prop_body,
        out_type=(
            jax.ShapeDtypeStruct((NQ, R, Q), jnp.float32),
            jax.ShapeDtypeStruct((NQ, R, Q), jnp.float32),
        ),
        mesh=_mesh(),
        compiler_params=_SC_PARAMS,
        scratch_types=[
            pltpu.VMEM((NCHUNK, CH), jnp.int32),
            pltpu.VMEM((NCHUNK, CH), jnp.int32),
            pltpu.VMEM((CH, Q), jnp.float32),
            pltpu.VMEM((CH, Q), jnp.float32),
            pltpu.VMEM((CH, Q), jnp.float32),
            pltpu.VMEM((CH, Q), jnp.float32),
            pltpu.VMEM((RPT, Q), jnp.float32),
            pltpu.VMEM((RPT, 16), jnp.float32),
            pltpu.VMEM((RPT, 16), jnp.float32),
            pltpu.SemaphoreType.DMA,
            pltpu.SemaphoreType.DMA,
            pltpu.SemaphoreType.DMA,
            pltpu.SemaphoreType.DMA,
            pltpu.SemaphoreType.DMA,
            pltpu.SemaphoreType.DMA,
            pltpu.SemaphoreType.DMA,
            pltpu.SemaphoreType.DMA,
            pltpu.VMEM_SHARED((R, Q), jnp.float32),
        ],
    )
    return f(h, n3, e3, z32, binv, dinv)


def _bn_body(p_ref, s_ref, t_ref, o_ref):
    o_ref[...] = jnp.maximum(p_ref[...] * s_ref[...] + t_ref[...], 0.0)


def _bn(p1, s1q, t1q):
    return pl.pallas_call(
        _bn_body,
        grid=(8,),
        in_specs=[
            pl.BlockSpec((NQ, R // 8, Q), lambda i: (0, i, 0)),
            pl.BlockSpec((NQ, 1, Q), lambda i: (0, 0, 0)),
            pl.BlockSpec((NQ, 1, Q), lambda i: (0, 0, 0)),
        ],
        out_specs=pl.BlockSpec((NQ, R // 8, Q), lambda i: (0, i, 0)),
        out_shape=jax.ShapeDtypeStruct((NQ, R, Q), jnp.float32),
    )(p1, s1q, t1q)


def _mm2_body(q_ref, w2_ref, s2_ref, t2_ref, wl_ref, bl_ref, o_ref, acc_ref):
    i = pl.program_id(0)

    @pl.when(i == 0)
    def _():
        acc_ref[...] = jnp.zeros_like(acc_ref)

    o = lax.dot_general(q_ref[0], w2_ref[0], (((1,), (1,)), ((), ())),
                        preferred_element_type=jnp.float32)
    for f in range(1, NQ):
        o = o + lax.dot_general(q_ref[f], w2_ref[f], (((1,), (1,)), ((), ())),
                                preferred_element_type=jnp.float32)
    o = jnp.maximum(o * s2_ref[...] + t2_ref[...], 0.0)
    acc_ref[...] += jnp.sum(o, axis=0, keepdims=True)

    @pl.when(i == pl.num_programs(0) - 1)
    def _():
        m = acc_ref[...] * (1.0 / N)
        o_ref[...] = lax.dot_general(m, wl_ref[...], (((1,), (1,)), ((), ())),
                                     preferred_element_type=jnp.float32
                                     ) + bl_ref[...]


def _mm2(q, w2q, s2, t2, wl, bl2):
    blk = 2000
    return pl.pallas_call(
        _mm2_body,
        grid=(N // blk,),
        in_specs=[
            pl.BlockSpec((NQ, blk, Q), lambda i: (0, i, 0)),
            pl.BlockSpec((NQ, D, Q), lambda i: (0, 0, 0)),
            pl.BlockSpec((1, D), lambda i: (0, 0)),
            pl.BlockSpec((1, D), lambda i: (0, 0)),
            pl.BlockSpec((D, D), lambda i: (0, 0)),
            pl.BlockSpec((1, D), lambda i: (0, 0)),
        ],
        out_specs=pl.BlockSpec((1, D), lambda i: (0, 0)),
        out_shape=jax.ShapeDtypeStruct((1, D), jnp.float32),
        scratch_shapes=[pltpu.VMEM((1, D), jnp.float32)],
    )(q, w2q, s2, t2, wl, bl2)


def kernel(x, hyperedge_index, W1, b1, g1, be1, W2, b2, g2, be2, Wl, bl):
    f32 = jnp.float32
    xp = jnp.concatenate([x, jnp.zeros((R - N, D), f32)], axis=0)
    w1q = W1.reshape(NQ, Q, D)

    ni = hyperedge_index[0]
    ei = hyperedge_index[1]
    n3 = jnp.full((TOT,), DUMMY, jnp.int32).at[:NNZ].set(ni).reshape(
        NS, NCHUNK, CH)
    e3 = jnp.full((TOT,), DUMMY, jnp.int32).at[:NNZ].set(ei).reshape(
        NS, NCHUNK, CH)

    z32 = jnp.zeros((RPT, Q), f32)
    z16 = jnp.zeros((RPT, 16), f32)
    ones = jnp.ones((CH, 16), f32)

    c0 = 1.0 / (1.0 + EPS) ** 0.5
    s1q = (g1 * c0).reshape(NQ, 1, Q)
    t1q = ((g1 * c0) * b1 + be1).reshape(NQ, 1, Q)
    s2 = (g2 * c0).reshape(1, D)
    t2 = ((g2 * c0) * b2 + be2).reshape(1, D)
    w2q = jnp.transpose(W2.reshape(D, NQ, Q), (1, 0, 2))

    binv, dinv = _deg(n3, e3, z16, ones)
    h1 = _mm1(xp, w1q)
    p1, _ = _prop(h1, n3, e3, z32, binv, dinv)
    z1 = _bn(p1, s1q, t1q)
    q, _ = _prop(z1, n3, e3, z32, binv, dinv)
    out = _mm2(q, w2q, s2, t2, Wl, bl.reshape(1, D))
    return out[0]
